# baseline (speedup 1.0000x reference)
import functools

import jax
import jax.numpy as jnp
from jax import lax
from jax.experimental import pallas as pl
from jax.experimental.pallas import tpu as pltpu
from jax.experimental.pallas import tpu_sc as plsc

L = 16
N = 32768
V = N // L
C = 4
Q = V // C
NBIN = 256 * L
R = 128
NW = 32
ROWS_PER_W = R // NW
MININT = -(2 ** 31)


def _flip(x):
    m = lax.shift_right_arithmetic(x, 31)
    return x ^ (m | MININT)


def _unflip_to_f32(k):
    m = lax.shift_right_arithmetic(k, 31)
    return plsc.bitcast(k ^ ((~m) | MININT), jnp.float32)


def _tile_body(a_hbm, b_hbm, out_hbm,
               buf_a, buf_b, buf_t, cnt0, cnt1, cnt2, cnt3, part, dma_sem):
    c = lax.axis_index("c")
    s = lax.axis_index("s")
    wid = s * 2 + c
    lane = lax.broadcasted_iota(jnp.int32, (L,), 0)
    ones = jnp.ones((L,), jnp.int32)
    sixteens = jnp.full((L,), L, jnp.int32)
    zero = jnp.zeros((L,), jnp.int32)
    cnt = (cnt0, cnt1, cnt2, cnt3)

    def zero_cnts():
        def body(i, _):
            for q in range(C):
                cnt[q][pl.ds(i * L, L)] = zero
            return 0
        lax.fori_loop(0, NBIN // L, body, 0, unroll=4)

    def digit_bin(k, shift):
        d = lax.shift_right_logical(k, shift) & 255
        return (d << 4) | lane

    def load_bins(src, v, shift, flip):
        ks, bs = [], []
        for q in range(C):
            k = src[pl.ds((q * Q + v) * L, L)]
            if flip:
                k = _flip(k)
            bs.append(digit_bin(k, shift))
            ks.append(k)
        return ks, bs

    def hist_pass(src, shift, flip):
        _, bs0 = load_bins(src, 0, shift, flip)

        def body(v, bs):
            nbs = load_bins(src, jnp.minimum(v + 1, Q - 1), shift, flip)[1]
            for q in range(C):
                plsc.addupdate_scatter(cnt[q], [bs[q]], ones)
            return tuple(nbs)
        lax.fori_loop(0, Q, body, tuple(bs0), unroll=4)

    def scan_pass():
        def body(i, carry):
            cs = [cnt[q][pl.ds(i * L, L)] for q in range(C)]
            t01 = cs[0] + cs[1]
            t012 = t01 + cs[2]
            tot = t012 + cs[3]
            incl = plsc.cumsum(tot)
            base = incl - tot + carry
            starts = (base, base + cs[0], base + t01, base + t012)
            for q in range(C):
                cnt[q][pl.ds(i * L, L)] = starts[q] << 4
            return carry + jnp.sum(tot)
        lax.fori_loop(0, NBIN // L, body, jnp.int32(0), unroll=2)

    def scatter_pass(src, dst, shift, flip=False, unflip_out=False):
        def prep(v):
            ks, bs = load_bins(src, v, shift, flip)
            if unflip_out:
                ks = [plsc.bitcast(_unflip_to_f32(k), jnp.int32) for k in ks]
            return ks, bs

        ks0, bs0 = prep(0)

        def body(v, carry):
            ks, bs = carry[:C], carry[C:]
            nks, nbs = prep(jnp.minimum(v + 1, Q - 1))
            poss = [plsc.load_gather(cnt[q], [bs[q]]) for q in range(C)]
            for q in range(C):
                plsc.addupdate_scatter(cnt[q], [bs[q]], sixteens)
            for q in range(C):
                p16 = poss[q]
                phys = (p16 & (N - L)) | lax.shift_right_logical(p16, 15)
                plsc.store_scatter(dst, [phys], ks[q])
            return tuple(nks) + tuple(nbs)
        lax.fori_loop(0, Q, body, tuple(ks0) + tuple(bs0), unroll=4)

    def scatter_acc_pass(src, sorted_a, shift, accs):
        ks0, bs0 = load_bins(src, 0, shift, False)

        def body(v, carry):
            accs, ks, bs = carry[:C], carry[C:2 * C], carry[2 * C:]
            nks, nbs = load_bins(src, jnp.minimum(v + 1, Q - 1), shift, False)
            poss = [plsc.load_gather(cnt[q], [bs[q]]) for q in range(C)]
            fas = []
            for q in range(C):
                p16 = poss[q]
                phys = (p16 & (N - L)) | lax.shift_right_logical(p16, 15)
                fas.append(plsc.bitcast(plsc.load_gather(sorted_a, [phys]),
                                        jnp.float32))
            for q in range(C):
                plsc.addupdate_scatter(cnt[q], [bs[q]], sixteens)
            out = []
            for q in range(C):
                d = fas[q] - _unflip_to_f32(ks[q])
                out.append(accs[q] + d * d)
            return tuple(out) + tuple(nks) + tuple(nbs)
        res = lax.fori_loop(0, Q, body, tuple(accs) + tuple(ks0) + tuple(bs0),
                            unroll=2)
        return res[:C]

    def sort_a():
        zero_cnts()
        hist_pass(buf_a, 0, True)
        scan_pass()
        scatter_pass(buf_a, buf_t, 0, flip=True)
        zero_cnts()
        hist_pass(buf_t, 8, False)
        scan_pass()
        scatter_pass(buf_t, buf_a, 8)
        zero_cnts()
        hist_pass(buf_a, 16, False)
        scan_pass()
        scatter_pass(buf_a, buf_t, 16)
        zero_cnts()
        hist_pass(buf_t, 24, False)
        scan_pass()
        scatter_pass(buf_t, buf_a, 24, unflip_out=True)

    def sort_b_acc(accs):
        zero_cnts()
        hist_pass(buf_b, 0, True)
        scan_pass()
        scatter_pass(buf_b, buf_t, 0, flip=True)
        zero_cnts()
        hist_pass(buf_t, 8, False)
        scan_pass()
        scatter_pass(buf_t, buf_b, 8)
        zero_cnts()
        hist_pass(buf_b, 16, False)
        scan_pass()
        scatter_pass(buf_b, buf_t, 16)
        zero_cnts()
        hist_pass(buf_t, 24, False)
        scan_pass()
        return scatter_acc_pass(buf_t, buf_a, 24, accs)

    def row_body(j, accs):
        base = (wid * ROWS_PER_W + j) * N
        pltpu.sync_copy(a_hbm.at[pl.ds(base, N)], buf_a)
        cp_b = pltpu.async_copy(b_hbm.at[pl.ds(base, N)], buf_b, dma_sem)
        sort_a()
        cp_b.wait()
        return sort_b_acc(accs)

    zf = jnp.zeros((L,), jnp.float32)
    accs = lax.fori_loop(0, ROWS_PER_W, row_body, (zf, zf, zf, zf))
    part[...] = accs[0] + accs[1] + accs[2] + accs[3]
    pltpu.sync_copy(part, out_hbm.at[pl.ds(wid * L, L)])


_sc_loss_partials = functools.partial(
    pl.kernel,
    out_type=jax.ShapeDtypeStruct((NW * L,), jnp.float32),
    mesh=plsc.VectorSubcoreMesh(core_axis_name="c", subcore_axis_name="s"),
    compiler_params=pltpu.CompilerParams(needs_layout_passes=False),
    scratch_types=[
        pltpu.VMEM((N,), jnp.int32),
        pltpu.VMEM((N,), jnp.int32),
        pltpu.VMEM((N,), jnp.int32),
        pltpu.VMEM((NBIN,), jnp.int32),
        pltpu.VMEM((NBIN,), jnp.int32),
        pltpu.VMEM((NBIN,), jnp.int32),
        pltpu.VMEM((NBIN,), jnp.int32),
        pltpu.VMEM((L,), jnp.float32),
        pltpu.SemaphoreType.DMA,
    ],
)(_tile_body)


@jax.jit
def kernel(psdA, psdB):
    i_a = lax.bitcast_convert_type(psdA, jnp.int32).reshape(-1)
    i_b = lax.bitcast_convert_type(psdB, jnp.int32).reshape(-1)
    partials = _sc_loss_partials(i_a, i_b)
    return jnp.sum(partials) * (1.0 / (R * N))


# speedup vs baseline: 1.0233x; 1.0233x over previous
import functools

import jax
import jax.numpy as jnp
from jax import lax
from jax.experimental import pallas as pl
from jax.experimental.pallas import tpu as pltpu
from jax.experimental.pallas import tpu_sc as plsc

L = 16
N = 32768
V = N // L
C = 4
Q = V // C
NBIN = 256 * L
R = 128
NW = 32
ROWS_PER_W = R // NW
MININT = -(2 ** 31)


def _flip(x):
    m = lax.shift_right_arithmetic(x, 31)
    return x ^ (m | MININT)


def _unflip_to_f32(k):
    m = lax.shift_right_arithmetic(k, 31)
    return plsc.bitcast(k ^ ((~m) | MININT), jnp.float32)


def _tile_body(a_hbm, b_hbm, out_hbm,
               buf_a, buf_b, buf_t, cnt0, cnt1, cnt2, cnt3, part, dma_sem):
    c = lax.axis_index("c")
    s = lax.axis_index("s")
    wid = s * 2 + c
    lane = lax.broadcasted_iota(jnp.int32, (L,), 0)
    ones = jnp.ones((L,), jnp.int32)
    sixteens = jnp.full((L,), L, jnp.int32)
    zero = jnp.zeros((L,), jnp.int32)
    cnt = (cnt0, cnt1, cnt2, cnt3)

    def zero_cnts():
        def body(i, _):
            for q in range(C):
                cnt[q][pl.ds(i * L, L)] = zero
            return 0
        lax.fori_loop(0, NBIN // L, body, 0, unroll=4)

    def digit_bin(k, shift):
        d = lax.shift_right_logical(k, shift) & 255
        return (d << 4) | lane

    def load_bins(src, v, shift, flip):
        ks, bs = [], []
        for q in range(C):
            k = src[pl.ds((q * Q + v) * L, L)]
            if flip:
                k = _flip(k)
            bs.append(digit_bin(k, shift))
            ks.append(k)
        return ks, bs

    def hist_pass(src, shift, flip):
        _, bs0 = load_bins(src, 0, shift, flip)

        def body(v, bs):
            nbs = load_bins(src, jnp.minimum(v + 1, Q - 1), shift, flip)[1]
            for q in range(C):
                plsc.addupdate_scatter(cnt[q], [bs[q]], ones)
            return tuple(nbs)
        lax.fori_loop(0, Q, body, tuple(bs0), unroll=4)

    def scan_pass():
        def body(i, carry):
            cs = [cnt[q][pl.ds(i * L, L)] for q in range(C)]
            t01 = cs[0] + cs[1]
            t012 = t01 + cs[2]
            tot = t012 + cs[3]
            incl = plsc.cumsum(tot)
            base = incl - tot + carry
            starts = (base, base + cs[0], base + t01, base + t012)
            for q in range(C):
                cnt[q][pl.ds(i * L, L)] = starts[q] << 4
            return carry + jnp.sum(tot)
        lax.fori_loop(0, NBIN // L, body, jnp.int32(0))

    def scatter_pass(src, dst, shift, flip=False, unflip_out=False):
        def prep(v):
            ks, bs = load_bins(src, v, shift, flip)
            if unflip_out:
                ks = [plsc.bitcast(_unflip_to_f32(k), jnp.int32) for k in ks]
            return ks, bs

        ks0, bs0 = prep(0)

        def body(v, carry):
            ks, bs = carry[:C], carry[C:]
            nks, nbs = prep(jnp.minimum(v + 1, Q - 1))
            poss = [plsc.load_gather(cnt[q], [bs[q]]) for q in range(C)]
            for q in range(C):
                plsc.addupdate_scatter(cnt[q], [bs[q]], sixteens)
            for q in range(C):
                p16 = poss[q]
                phys = (p16 & (N - L)) | lax.shift_right_logical(p16, 15)
                plsc.store_scatter(dst, [phys], ks[q])
            return tuple(nks) + tuple(nbs)
        lax.fori_loop(0, Q, body, tuple(ks0) + tuple(bs0), unroll=4)

    def scatter_acc_pass(src, sorted_a, shift, accs):
        ks0, bs0 = load_bins(src, 0, shift, False)

        def body(v, carry):
            accs, ks, bs = carry[:C], carry[C:2 * C], carry[2 * C:]
            nks, nbs = load_bins(src, jnp.minimum(v + 1, Q - 1), shift, False)
            poss = [plsc.load_gather(cnt[q], [bs[q]]) for q in range(C)]
            fas = []
            for q in range(C):
                p16 = poss[q]
                phys = (p16 & (N - L)) | lax.shift_right_logical(p16, 15)
                fas.append(plsc.bitcast(plsc.load_gather(sorted_a, [phys]),
                                        jnp.float32))
            for q in range(C):
                plsc.addupdate_scatter(cnt[q], [bs[q]], sixteens)
            out = []
            for q in range(C):
                d = fas[q] - _unflip_to_f32(ks[q])
                out.append(accs[q] + d * d)
            return tuple(out) + tuple(nks) + tuple(nbs)
        res = lax.fori_loop(0, Q, body, tuple(accs) + tuple(ks0) + tuple(bs0),
                            unroll=2)
        return res[:C]

    def sort_a():
        zero_cnts()
        hist_pass(buf_a, 0, True)
        scan_pass()
        scatter_pass(buf_a, buf_t, 0, flip=True)
        zero_cnts()
        hist_pass(buf_t, 8, False)
        scan_pass()
        scatter_pass(buf_t, buf_a, 8)
        zero_cnts()
        hist_pass(buf_a, 16, False)
        scan_pass()
        scatter_pass(buf_a, buf_t, 16)
        zero_cnts()
        hist_pass(buf_t, 24, False)
        scan_pass()
        scatter_pass(buf_t, buf_a, 24, unflip_out=True)

    def sort_b_acc(accs):
        zero_cnts()
        hist_pass(buf_b, 0, True)
        scan_pass()
        scatter_pass(buf_b, buf_t, 0, flip=True)
        zero_cnts()
        hist_pass(buf_t, 8, False)
        scan_pass()
        scatter_pass(buf_t, buf_b, 8)
        zero_cnts()
        hist_pass(buf_b, 16, False)
        scan_pass()
        scatter_pass(buf_b, buf_t, 16)
        zero_cnts()
        hist_pass(buf_t, 24, False)
        scan_pass()
        return scatter_acc_pass(buf_t, buf_a, 24, accs)

    def row_body(j, accs):
        base = (wid * ROWS_PER_W + j) * N
        pltpu.sync_copy(a_hbm.at[pl.ds(base, N)], buf_a)
        cp_b = pltpu.async_copy(b_hbm.at[pl.ds(base, N)], buf_b, dma_sem)
        sort_a()
        cp_b.wait()
        return sort_b_acc(accs)

    zf = jnp.zeros((L,), jnp.float32)
    accs = lax.fori_loop(0, ROWS_PER_W, row_body, (zf, zf, zf, zf))
    part[...] = accs[0] + accs[1] + accs[2] + accs[3]
    pltpu.sync_copy(part, out_hbm.at[pl.ds(wid * L, L)])


_sc_loss_partials = functools.partial(
    pl.kernel,
    out_type=jax.ShapeDtypeStruct((NW * L,), jnp.float32),
    mesh=plsc.VectorSubcoreMesh(core_axis_name="c", subcore_axis_name="s"),
    compiler_params=pltpu.CompilerParams(needs_layout_passes=False),
    scratch_types=[
        pltpu.VMEM((N,), jnp.int32),
        pltpu.VMEM((N,), jnp.int32),
        pltpu.VMEM((N,), jnp.int32),
        pltpu.VMEM((NBIN,), jnp.int32),
        pltpu.VMEM((NBIN,), jnp.int32),
        pltpu.VMEM((NBIN,), jnp.int32),
        pltpu.VMEM((NBIN,), jnp.int32),
        pltpu.VMEM((L,), jnp.float32),
        pltpu.SemaphoreType.DMA,
    ],
)(_tile_body)


@jax.jit
def kernel(psdA, psdB):
    i_a = lax.bitcast_convert_type(psdA, jnp.int32).reshape(-1)
    i_b = lax.bitcast_convert_type(psdB, jnp.int32).reshape(-1)
    partials = _sc_loss_partials(i_a, i_b)
    return jnp.sum(partials) * (1.0 / (R * N))


# speedup vs baseline: 1.1022x; 1.0771x over previous
import functools

import jax
import jax.numpy as jnp
from jax import lax
from jax.experimental import pallas as pl
from jax.experimental.pallas import tpu as pltpu
from jax.experimental.pallas import tpu_sc as plsc

L = 16
N = 32768
V = N // L
C = 4
Q = V // C
NBIN = 256 * L
R = 128
NW = 32
ROWS_PER_W = R // NW
MININT = -(2 ** 31)


def _flip(x):
    m = lax.shift_right_arithmetic(x, 31)
    return x ^ (m | MININT)


def _unflip_to_f32(k):
    m = lax.shift_right_arithmetic(k, 31)
    return plsc.bitcast(k ^ ((~m) | MININT), jnp.float32)


def _tile_body(a_hbm, b_hbm, out_hbm,
               buf_a, buf_b, buf_t, cnt0, cnt1, cnt2, cnt3, part, dma_sem):
    c = lax.axis_index("c")
    s = lax.axis_index("s")
    wid = s * 2 + c
    lane = lax.broadcasted_iota(jnp.int32, (L,), 0)
    ones = jnp.ones((L,), jnp.int32)
    sixteens = jnp.full((L,), L, jnp.int32)
    zero = jnp.zeros((L,), jnp.int32)
    cnt = (cnt0, cnt1, cnt2, cnt3)

    def zero_cnts():
        def body(i, _):
            for q in range(C):
                cnt[q][pl.ds(i * L, L)] = zero
            return 0
        lax.fori_loop(0, NBIN // L, body, 0, unroll=4)

    def digit_bin(k, shift):
        d = lax.shift_right_logical(k, shift) & 255
        return (d << 4) | lane

    def load_bins(src, v, shift, flip):
        ks, bs = [], []
        for q in range(C):
            k = src[pl.ds((q * Q + v) * L, L)]
            if flip:
                k = _flip(k)
            bs.append(digit_bin(k, shift))
            ks.append(k)
        return ks, bs

    def hist_pass(src, shift, flip):
        _, bs0 = load_bins(src, 0, shift, flip)

        def body(v, bs):
            nbs = load_bins(src, jnp.minimum(v + 1, Q - 1), shift, flip)[1]
            for q in range(C):
                plsc.addupdate_scatter(cnt[q], [bs[q]], ones)
            return tuple(nbs)
        lax.fori_loop(0, Q, body, tuple(bs0), unroll=4)

    def scan_pass():
        def body(i, carry):
            cs = [cnt[q][pl.ds(i * L, L)] for q in range(C)]
            t01 = cs[0] + cs[1]
            t012 = t01 + cs[2]
            tot = t012 + cs[3]
            incl = plsc.cumsum(tot)
            base = incl - tot + carry
            starts = (base, base + cs[0], base + t01, base + t012)
            for q in range(C):
                cnt[q][pl.ds(i * L, L)] = starts[q] << 4
            return carry + jnp.sum(tot)
        lax.fori_loop(0, NBIN // L, body, jnp.int32(0))

    def scatter_pass(src, dst, shift, flip=False, unflip_out=False):
        def prep(v):
            ks, bs = load_bins(src, v, shift, flip)
            if unflip_out:
                ks = [plsc.bitcast(_unflip_to_f32(k), jnp.int32) for k in ks]
            return ks, bs

        ks0, bs0 = prep(0)

        def body(v, carry):
            ks, bs = carry[:C], carry[C:]
            nks, nbs = prep(jnp.minimum(v + 1, Q - 1))
            poss = [plsc.load_gather(cnt[q], [bs[q]]) for q in range(C)]
            for q in range(C):
                plsc.addupdate_scatter(cnt[q], [bs[q]], sixteens)
            for q in range(C):
                p16 = poss[q]
                phys = (p16 & (N - L)) | lax.shift_right_logical(p16, 15)
                plsc.store_scatter(dst, [phys], ks[q])
            return tuple(nks) + tuple(nbs)
        lax.fori_loop(0, Q, body, tuple(ks0) + tuple(bs0), unroll=4)

    def scatter_acc_pass(src, sorted_a, shift, accs):
        ks0, bs0 = load_bins(src, 0, shift, False)

        def body(v, carry):
            accs, ks, bs = carry[:C], carry[C:2 * C], carry[2 * C:]
            nks, nbs = load_bins(src, jnp.minimum(v + 1, Q - 1), shift, False)
            poss = [plsc.load_gather(cnt[q], [bs[q]]) for q in range(C)]
            fas = []
            for q in range(C):
                p16 = poss[q]
                phys = (p16 & (N - L)) | lax.shift_right_logical(p16, 15)
                fas.append(plsc.bitcast(plsc.load_gather(sorted_a, [phys]),
                                        jnp.float32))
            for q in range(C):
                plsc.addupdate_scatter(cnt[q], [bs[q]], sixteens)
            out = []
            for q in range(C):
                d = fas[q] - _unflip_to_f32(ks[q])
                out.append(accs[q] + d * d)
            return tuple(out) + tuple(nks) + tuple(nbs)
        res = lax.fori_loop(0, Q, body, tuple(accs) + tuple(ks0) + tuple(bs0),
                            unroll=2)
        return res[:C]

    def sort_a():
        zero_cnts()
        hist_pass(buf_a, 0, True)
        scan_pass()
        scatter_pass(buf_a, buf_t, 0, flip=True)
        zero_cnts()
        hist_pass(buf_t, 8, False)
        scan_pass()
        scatter_pass(buf_t, buf_a, 8)
        zero_cnts()
        hist_pass(buf_a, 16, False)
        scan_pass()
        scatter_pass(buf_a, buf_t, 16)
        zero_cnts()
        hist_pass(buf_t, 24, False)
        scan_pass()
        scatter_pass(buf_t, buf_a, 24, unflip_out=True)

    def sort_b_acc(accs):
        zero_cnts()
        hist_pass(buf_b, 0, True)
        scan_pass()
        scatter_pass(buf_b, buf_t, 0, flip=True)
        zero_cnts()
        hist_pass(buf_t, 8, False)
        scan_pass()
        scatter_pass(buf_t, buf_b, 8)
        zero_cnts()
        hist_pass(buf_b, 16, False)
        scan_pass()
        scatter_pass(buf_b, buf_t, 16)
        zero_cnts()
        hist_pass(buf_t, 24, False)
        scan_pass()
        return scatter_acc_pass(buf_t, buf_a, 24, accs)

    def row_body(j, accs):
        row = wid * ROWS_PER_W + j
        pltpu.sync_copy(a_hbm.at[row], buf_a)
        cp_b = pltpu.async_copy(b_hbm.at[row], buf_b, dma_sem)
        sort_a()
        cp_b.wait()
        return sort_b_acc(accs)

    zf = jnp.zeros((L,), jnp.float32)
    accs = lax.fori_loop(0, ROWS_PER_W, row_body, (zf, zf, zf, zf))
    part[...] = accs[0] + accs[1] + accs[2] + accs[3]
    pltpu.sync_copy(part, out_hbm.at[pl.ds(wid * L, L)])


_sc_loss_partials = functools.partial(
    pl.kernel,
    out_type=jax.ShapeDtypeStruct((NW * L,), jnp.float32),
    mesh=plsc.VectorSubcoreMesh(core_axis_name="c", subcore_axis_name="s"),
    compiler_params=pltpu.CompilerParams(needs_layout_passes=False),
    scratch_types=[
        pltpu.VMEM((N,), jnp.int32),
        pltpu.VMEM((N,), jnp.int32),
        pltpu.VMEM((N,), jnp.int32),
        pltpu.VMEM((NBIN,), jnp.int32),
        pltpu.VMEM((NBIN,), jnp.int32),
        pltpu.VMEM((NBIN,), jnp.int32),
        pltpu.VMEM((NBIN,), jnp.int32),
        pltpu.VMEM((L,), jnp.float32),
        pltpu.SemaphoreType.DMA,
    ],
)(_tile_body)


@jax.jit
def kernel(psdA, psdB):
    i_a = lax.bitcast_convert_type(psdA, jnp.int32)
    i_b = lax.bitcast_convert_type(psdB, jnp.int32)
    partials = _sc_loss_partials(i_a, i_b)
    return jnp.sum(partials) * (1.0 / (R * N))


# speedup vs baseline: 1.1210x; 1.0171x over previous
import functools

import jax
import jax.numpy as jnp
from jax import lax
from jax.experimental import pallas as pl
from jax.experimental.pallas import tpu as pltpu
from jax.experimental.pallas import tpu_sc as plsc

L = 16
N = 32768
V = N // L
C = 4
Q = V // C
NBIN = 256 * L
R = 128
NW = 32
ROWS_PER_W = R // NW
MININT = -(2 ** 31)


def _flip(x):
    m = lax.shift_right_arithmetic(x, 31)
    return x ^ (m | MININT)


def _unflip_to_f32(k):
    m = lax.shift_right_arithmetic(k, 31)
    return plsc.bitcast(k ^ ((~m) | MININT), jnp.float32)


def _tile_body(a_hbm, b_hbm, out_hbm,
               buf_a, buf_b, buf_t, cnt0, cnt1, cnt2, cnt3, part, dma_sem):
    c = lax.axis_index("c")
    s = lax.axis_index("s")
    wid = s * 2 + c
    lane = lax.broadcasted_iota(jnp.int32, (L,), 0)
    ones = jnp.ones((L,), jnp.int32)
    sixteens = jnp.full((L,), L, jnp.int32)
    zero = jnp.zeros((L,), jnp.int32)
    cnt = (cnt0, cnt1, cnt2, cnt3)

    def zero_cnts():
        def body(i, _):
            for q in range(C):
                cnt[q][pl.ds(i * L, L)] = zero
            return 0
        lax.fori_loop(0, NBIN // L, body, 0, unroll=4)

    def digit_bin(k, shift):
        d = lax.shift_right_logical(k, shift) & 255
        return (d << 4) | lane

    def load_bins(src, v, shift, flip):
        ks, bs = [], []
        for q in range(C):
            k = src[pl.ds((q * Q + v) * L, L)]
            if flip:
                k = _flip(k)
            bs.append(digit_bin(k, shift))
            ks.append(k)
        return ks, bs

    def hist_pass(src, shift, flip):
        _, bs0 = load_bins(src, 0, shift, flip)

        def body(v, bs):
            nbs = load_bins(src, jnp.minimum(v + 1, Q - 1), shift, flip)[1]
            for q in range(C):
                plsc.addupdate_scatter(cnt[q], [bs[q]], ones)
            return tuple(nbs)
        lax.fori_loop(0, Q, body, tuple(bs0), unroll=8)

    def scan_pass():
        def body(i, carry):
            cs = [cnt[q][pl.ds(i * L, L)] for q in range(C)]
            t01 = cs[0] + cs[1]
            t012 = t01 + cs[2]
            tot = t012 + cs[3]
            incl = plsc.cumsum(tot)
            base = incl - tot + carry
            starts = (base, base + cs[0], base + t01, base + t012)
            for q in range(C):
                cnt[q][pl.ds(i * L, L)] = starts[q] << 4
            return carry + jnp.sum(tot)
        lax.fori_loop(0, NBIN // L, body, jnp.int32(0))

    def scatter_pass(src, dst, shift, flip=False, unflip_out=False):
        def prep(v):
            ks, bs = load_bins(src, v, shift, flip)
            if unflip_out:
                ks = [plsc.bitcast(_unflip_to_f32(k), jnp.int32) for k in ks]
            return ks, bs

        ks0, bs0 = prep(0)

        def body(v, carry):
            ks, bs = carry[:C], carry[C:]
            nks, nbs = prep(jnp.minimum(v + 1, Q - 1))
            poss = [plsc.load_gather(cnt[q], [bs[q]]) for q in range(C)]
            for q in range(C):
                plsc.addupdate_scatter(cnt[q], [bs[q]], sixteens)
            for q in range(C):
                p16 = poss[q]
                phys = (p16 & (N - L)) | lax.shift_right_logical(p16, 15)
                plsc.store_scatter(dst, [phys], ks[q])
            return tuple(nks) + tuple(nbs)
        lax.fori_loop(0, Q, body, tuple(ks0) + tuple(bs0), unroll=8)

    def scatter_acc_pass(src, sorted_a, shift, accs):
        ks0, bs0 = load_bins(src, 0, shift, False)

        def body(v, carry):
            accs, ks, bs = carry[:C], carry[C:2 * C], carry[2 * C:]
            nks, nbs = load_bins(src, jnp.minimum(v + 1, Q - 1), shift, False)
            poss = [plsc.load_gather(cnt[q], [bs[q]]) for q in range(C)]
            fas = []
            for q in range(C):
                p16 = poss[q]
                phys = (p16 & (N - L)) | lax.shift_right_logical(p16, 15)
                fas.append(plsc.bitcast(plsc.load_gather(sorted_a, [phys]),
                                        jnp.float32))
            for q in range(C):
                plsc.addupdate_scatter(cnt[q], [bs[q]], sixteens)
            out = []
            for q in range(C):
                d = fas[q] - _unflip_to_f32(ks[q])
                out.append(accs[q] + d * d)
            return tuple(out) + tuple(nks) + tuple(nbs)
        res = lax.fori_loop(0, Q, body, tuple(accs) + tuple(ks0) + tuple(bs0),
                            unroll=4)
        return res[:C]

    def sort_a():
        zero_cnts()
        hist_pass(buf_a, 0, True)
        scan_pass()
        scatter_pass(buf_a, buf_t, 0, flip=True)
        zero_cnts()
        hist_pass(buf_t, 8, False)
        scan_pass()
        scatter_pass(buf_t, buf_a, 8)
        zero_cnts()
        hist_pass(buf_a, 16, False)
        scan_pass()
        scatter_pass(buf_a, buf_t, 16)
        zero_cnts()
        hist_pass(buf_t, 24, False)
        scan_pass()
        scatter_pass(buf_t, buf_a, 24, unflip_out=True)

    def sort_b_acc(accs):
        zero_cnts()
        hist_pass(buf_b, 0, True)
        scan_pass()
        scatter_pass(buf_b, buf_t, 0, flip=True)
        zero_cnts()
        hist_pass(buf_t, 8, False)
        scan_pass()
        scatter_pass(buf_t, buf_b, 8)
        zero_cnts()
        hist_pass(buf_b, 16, False)
        scan_pass()
        scatter_pass(buf_b, buf_t, 16)
        zero_cnts()
        hist_pass(buf_t, 24, False)
        scan_pass()
        return scatter_acc_pass(buf_t, buf_a, 24, accs)

    def row_body(j, accs):
        row = wid * ROWS_PER_W + j
        pltpu.sync_copy(a_hbm.at[row], buf_a)
        cp_b = pltpu.async_copy(b_hbm.at[row], buf_b, dma_sem)
        sort_a()
        cp_b.wait()
        return sort_b_acc(accs)

    zf = jnp.zeros((L,), jnp.float32)
    accs = lax.fori_loop(0, ROWS_PER_W, row_body, (zf, zf, zf, zf))
    part[...] = accs[0] + accs[1] + accs[2] + accs[3]
    pltpu.sync_copy(part, out_hbm.at[pl.ds(wid * L, L)])


_sc_loss_partials = functools.partial(
    pl.kernel,
    out_type=jax.ShapeDtypeStruct((NW * L,), jnp.float32),
    mesh=plsc.VectorSubcoreMesh(core_axis_name="c", subcore_axis_name="s"),
    compiler_params=pltpu.CompilerParams(needs_layout_passes=False),
    scratch_types=[
        pltpu.VMEM((N,), jnp.int32),
        pltpu.VMEM((N,), jnp.int32),
        pltpu.VMEM((N,), jnp.int32),
        pltpu.VMEM((NBIN,), jnp.int32),
        pltpu.VMEM((NBIN,), jnp.int32),
        pltpu.VMEM((NBIN,), jnp.int32),
        pltpu.VMEM((NBIN,), jnp.int32),
        pltpu.VMEM((L,), jnp.float32),
        pltpu.SemaphoreType.DMA,
    ],
)(_tile_body)


@jax.jit
def kernel(psdA, psdB):
    i_a = lax.bitcast_convert_type(psdA, jnp.int32)
    i_b = lax.bitcast_convert_type(psdB, jnp.int32)
    partials = _sc_loss_partials(i_a, i_b)
    return jnp.sum(partials) * (1.0 / (R * N))
